# baseline (speedup 1.0000x reference)
import functools

import jax
import jax.numpy as jnp
from jax import lax
from jax.experimental import pallas as pl
from jax.experimental.pallas import tpu as pltpu
from jax.experimental.pallas import tpu_sc as plsc

_VOCAB = 100000
_EMBED = 128
_OUT = 2
_SEQ = 200
_BATCH = 4096

_NTILES = 32
_BPT = _BATCH // _NTILES
_SEQ1 = 104
_SEQ2 = _SEQ - _SEQ1


def _project_pack(table, w):
    vb = 20480
    grid = (pl.cdiv(_VOCAB, vb),)

    def body(tab_ref, w_ref, out_ref):
        wb = (w_ref[...] * (1.0 / _SEQ)).astype(jnp.bfloat16)
        tb = tab_ref[...].astype(jnp.bfloat16)
        res = lax.dot_general(
            wb, tb,
            dimension_numbers=(((1,), (1,)), ((), ())),
            preferred_element_type=jnp.float32,
        )
        u = lax.bitcast_convert_type(res, jnp.uint32)
        packed = (u[0] >> 16) | (u[1] & jnp.uint32(0xFFFF0000))
        out_ref[...] = lax.bitcast_convert_type(packed, jnp.int32)

    return pl.pallas_call(
        body,
        grid=grid,
        in_specs=[
            pl.BlockSpec((vb, _EMBED), lambda j: (j, 0)),
            pl.BlockSpec((_OUT, _EMBED), lambda j: (0, 0)),
        ],
        out_specs=pl.BlockSpec((vb,), lambda j: (j,)),
        out_shape=jax.ShapeDtypeStruct((_VOCAB,), jnp.int32),
    )(table, w)


def _sc_pool(packed, text_i32):
    mesh = plsc.VectorSubcoreMesh(core_axis_name="c", subcore_axis_name="s")

    @functools.partial(
        pl.kernel,
        mesh=mesh,
        compiler_params=pltpu.CompilerParams(needs_layout_passes=False,
                                             use_tc_tiling_on_sc=False),
        out_type=[
            jax.ShapeDtypeStruct((_BATCH,), jnp.float32),
            jax.ShapeDtypeStruct((_BATCH,), jnp.float32),
        ],
        scratch_types=[
            pltpu.VMEM((_VOCAB,), jnp.int32),
            pltpu.VMEM((_SEQ1, _BPT), jnp.int32),
            pltpu.VMEM((_BPT,), jnp.float32),
            pltpu.VMEM((_BPT,), jnp.float32),
            pltpu.VMEM_SHARED((_VOCAB,), jnp.int32),
            pltpu.SemaphoreType.DMA,
            pltpu.SemaphoreType.DMA,
        ],
    )
    def k(p_hbm, t_hbm, out0_hbm, out1_hbm, p_v, t_v, o0_v, o1_v, p_sh,
          sem_p, sem_t):
        c = lax.axis_index("c")
        s = lax.axis_index("s")
        wid = s * 2 + c
        base = pl.multiple_of(wid * _BPT, _BPT)

        ct = pltpu.make_async_copy(
            t_hbm.at[pl.ds(0, _SEQ1), pl.ds(base, _BPT)], t_v, sem_t)
        ct.start()

        @pl.when(s == 0)
        def _():
            pltpu.sync_copy(p_hbm, p_sh)

        plsc.subcore_barrier()
        cp = pltpu.make_async_copy(p_sh, p_v, sem_p)
        cp.start()
        cp.wait()
        ct.wait()

        zero = jnp.zeros((16,), jnp.float32)
        himask = jnp.full((16,), -65536, jnp.int32)

        def make_body(g):
            def body(i, accs):
                a0, a1 = accs
                idx = t_v[i, pl.ds(g * 16, 16)]
                word = plsc.load_gather(p_v, [idx])
                a0 = a0 + plsc.bitcast(word << 16, jnp.float32)
                a1 = a1 + plsc.bitcast(word & himask, jnp.float32)
                return (a0, a1)
            return body

        def gloop1(g, carry):
            a0, a1 = plsc.parallel_loop(0, _SEQ1, unroll=8,
                                        carry=(zero, zero))(make_body(g))
            o0_v[pl.ds(g * 16, 16)] = a0
            o1_v[pl.ds(g * 16, 16)] = a1
            return carry

        lax.fori_loop(0, _BPT // 16, gloop1, 0)

        ct2 = pltpu.make_async_copy(
            t_hbm.at[pl.ds(_SEQ1, _SEQ2), pl.ds(base, _BPT)],
            t_v.at[pl.ds(0, _SEQ2)], sem_t)
        ct2.start()
        ct2.wait()

        def gloop2(g, carry):
            a0, a1 = plsc.parallel_loop(0, _SEQ2, unroll=8,
                                        carry=(zero, zero))(make_body(g))
            o0_v[pl.ds(g * 16, 16)] = o0_v[pl.ds(g * 16, 16)] + a0
            o1_v[pl.ds(g * 16, 16)] = o1_v[pl.ds(g * 16, 16)] + a1
            return carry

        lax.fori_loop(0, _BPT // 16, gloop2, 0)

        pltpu.sync_copy(o0_v, out0_hbm.at[pl.ds(base, _BPT)])
        pltpu.sync_copy(o1_v, out1_hbm.at[pl.ds(base, _BPT)])

    return k(packed, text_i32)


def kernel(text, table, W, b):
    packed = _project_pack(table, W.astype(jnp.float32))
    out0, out1 = _sc_pool(packed, text.astype(jnp.int32))
    return jnp.stack([out0, out1], axis=1) + b.astype(jnp.float32)


# speedup vs baseline: 1.0846x; 1.0846x over previous
import functools

import jax
import jax.numpy as jnp
from jax import lax
from jax.experimental import pallas as pl
from jax.experimental.pallas import tpu as pltpu
from jax.experimental.pallas import tpu_sc as plsc

_VOCAB = 100000
_EMBED = 128
_OUT = 2
_SEQ = 200
_BATCH = 4096

_NTILES = 32
_BPT = _BATCH // _NTILES
_SEQ1 = 104
_SEQ2 = _SEQ - _SEQ1


def _project_pack(table, w):
    vb = 20480
    grid = (pl.cdiv(_VOCAB, vb),)

    def body(tab_ref, w_ref, out_ref):
        wb = (w_ref[...] * (1.0 / _SEQ)).astype(jnp.bfloat16)
        tb = tab_ref[...].astype(jnp.bfloat16)
        res = lax.dot_general(
            wb, tb,
            dimension_numbers=(((1,), (1,)), ((), ())),
            preferred_element_type=jnp.float32,
        )
        u = lax.bitcast_convert_type(res, jnp.uint32)
        packed = (u[0] >> 16) | (u[1] & jnp.uint32(0xFFFF0000))
        out_ref[...] = lax.bitcast_convert_type(packed, jnp.int32)

    return pl.pallas_call(
        body,
        grid=grid,
        in_specs=[
            pl.BlockSpec((vb, _EMBED), lambda j: (j, 0)),
            pl.BlockSpec((_OUT, _EMBED), lambda j: (0, 0)),
        ],
        out_specs=pl.BlockSpec((vb,), lambda j: (j,)),
        out_shape=jax.ShapeDtypeStruct((_VOCAB,), jnp.int32),
    )(table, w)


def _sc_pool(packed, text_i32):
    mesh = plsc.VectorSubcoreMesh(core_axis_name="c", subcore_axis_name="s")

    @functools.partial(
        pl.kernel,
        mesh=mesh,
        compiler_params=pltpu.CompilerParams(needs_layout_passes=False),
        out_type=[
            jax.ShapeDtypeStruct((_BATCH,), jnp.float32),
            jax.ShapeDtypeStruct((_BATCH,), jnp.float32),
        ],
        scratch_types=[
            pltpu.VMEM((_VOCAB,), jnp.int32),
            pltpu.VMEM((_SEQ1, _BPT), jnp.int32),
            pltpu.VMEM((_BPT,), jnp.float32),
            pltpu.VMEM((_BPT,), jnp.float32),
            pltpu.VMEM_SHARED((_VOCAB,), jnp.int32),
            pltpu.SemaphoreType.DMA,
            pltpu.SemaphoreType.DMA,
        ],
    )
    def k(p_hbm, t_hbm, out0_hbm, out1_hbm, p_v, t_v, o0_v, o1_v, p_sh,
          sem_p, sem_t):
        c = lax.axis_index("c")
        s = lax.axis_index("s")
        wid = s * 2 + c
        base = pl.multiple_of(wid * _BPT, _BPT)

        ct = pltpu.make_async_copy(
            t_hbm.at[pl.ds(0, _SEQ1), pl.ds(base, _BPT)], t_v, sem_t)
        ct.start()

        @pl.when(s == 0)
        def _():
            pltpu.sync_copy(p_hbm, p_sh)

        plsc.subcore_barrier()
        cp = pltpu.make_async_copy(p_sh, p_v, sem_p)
        cp.start()
        cp.wait()
        ct.wait()

        zero = jnp.zeros((16,), jnp.float32)
        himask = jnp.full((16,), -65536, jnp.int32)

        def make_body(g):
            def body(i, accs):
                a0, a1 = accs
                idx = t_v[i, pl.ds(g * 16, 16)]
                word = plsc.load_gather(p_v, [idx])
                a0 = a0 + plsc.bitcast(word << 16, jnp.float32)
                a1 = a1 + plsc.bitcast(word & himask, jnp.float32)
                return (a0, a1)
            return body

        def gloop1(g, carry):
            a0, a1 = plsc.parallel_loop(0, _SEQ1, unroll=8,
                                        carry=(zero, zero))(make_body(g))
            o0_v[pl.ds(g * 16, 16)] = a0
            o1_v[pl.ds(g * 16, 16)] = a1
            return carry

        lax.fori_loop(0, _BPT // 16, gloop1, 0)

        ct2 = pltpu.make_async_copy(
            t_hbm.at[pl.ds(_SEQ1, _SEQ2), pl.ds(base, _BPT)],
            t_v.at[pl.ds(0, _SEQ2)], sem_t)
        ct2.start()
        ct2.wait()

        def gloop2(g, carry):
            a0, a1 = plsc.parallel_loop(0, _SEQ2, unroll=8,
                                        carry=(zero, zero))(make_body(g))
            o0_v[pl.ds(g * 16, 16)] = o0_v[pl.ds(g * 16, 16)] + a0
            o1_v[pl.ds(g * 16, 16)] = o1_v[pl.ds(g * 16, 16)] + a1
            return carry

        lax.fori_loop(0, _BPT // 16, gloop2, 0)

        pltpu.sync_copy(o0_v, out0_hbm.at[pl.ds(base, _BPT)])
        pltpu.sync_copy(o1_v, out1_hbm.at[pl.ds(base, _BPT)])

    return k(packed, text_i32)


def kernel(text, table, W, b):
    packed = _project_pack(table, W.astype(jnp.float32))
    out0, out1 = _sc_pool(packed, text.astype(jnp.int32))
    return jnp.stack([out0, out1], axis=1) + b.astype(jnp.float32)
